# baseline (speedup 1.0000x reference)
import functools

import jax
import jax.numpy as jnp
from jax import lax
from jax.experimental import pallas as pl
from jax.experimental.pallas import tpu as pltpu
from jax.experimental.pallas import tpu_sc as plsc

_CH = 2
_NBUF = 3


def _make_sc_kernel(batch, dim):
    info = plsc.get_sparse_core_info()
    nc, ns, lanes = info.num_cores, info.num_subcores, info.num_lanes
    nw = nc * ns
    bpw = batch // nw
    assert batch % nw == 0 and dim % lanes == 0 and bpw % _CH == 0
    nchunk = dim // lanes
    msize = dim * dim

    mesh = plsc.VectorSubcoreMesh(core_axis_name="c", subcore_axis_name="s")

    @functools.partial(
        pl.kernel,
        mesh=mesh,
        out_type=jax.ShapeDtypeStruct((batch * msize,), jnp.float32),
        scratch_types=[
            pltpu.VMEM((bpw, dim), jnp.float32),
        ] + [pltpu.VMEM((_CH * msize,), jnp.float32) for _ in range(_NBUF)]
          + [pltpu.SemaphoreType.DMA for _ in range(_NBUF)],
        compiler_params=pltpu.CompilerParams(needs_layout_passes=False),
    )
    def diag_sc(e_hbm, z_hbm, out_hbm, ebuf, *rest):
        bufs = rest[:_NBUF]
        sems = rest[_NBUF:]
        wid = lax.axis_index("s") * nc + lax.axis_index("c")
        base = wid * bpw
        pltpu.sync_copy(e_hbm.at[pl.ds(base, bpw)], ebuf)
        for buf in bufs:
            pltpu.sync_copy(z_hbm, buf)
        handles = [None] * _NBUF
        for c in range(bpw // _CH):
            s = c % _NBUF
            buf = bufs[s]
            if handles[s] is not None:
                handles[s].wait()
            for m in range(_CH):
                b = c * _CH + m
                for k in range(nchunk):
                    idx = (jnp.arange(lanes, dtype=jnp.int32)
                           + (k * lanes)) * (dim + 1) + (m * msize)
                    vals = jnp.exp(ebuf[b, pl.ds(k * lanes, lanes)])
                    plsc.store_scatter(buf, [idx], vals)
            dst = out_hbm.at[pl.ds((base + c * _CH) * msize, _CH * msize)]
            handles[s] = pltpu.async_copy(buf, dst, sems[s])
        for h in handles:
            if h is not None:
                h.wait()

    return diag_sc


def kernel(embeddings):
    batch, dim, _ = embeddings.shape
    e2 = embeddings[:, :, 0]
    zeros = jnp.zeros((_CH * dim * dim,), dtype=jnp.float32)
    sc = _make_sc_kernel(batch, dim)
    return sc(e2, zeros).reshape(batch, dim, dim)


# speedup vs baseline: 1.0550x; 1.0550x over previous
import functools

import jax
import jax.numpy as jnp
from jax import lax
from jax.experimental import pallas as pl
from jax.experimental.pallas import tpu as pltpu
from jax.experimental.pallas import tpu_sc as plsc

_CH = 3
_NBUF = 2


def _make_sc_kernel(batch, dim):
    info = plsc.get_sparse_core_info()
    nc, ns, lanes = info.num_cores, info.num_subcores, info.num_lanes
    nw = nc * ns
    bpw = batch // nw
    assert batch % nw == 0 and dim % lanes == 0
    nchunk = dim // lanes
    msize = dim * dim

    mesh = plsc.VectorSubcoreMesh(core_axis_name="c", subcore_axis_name="s")

    @functools.partial(
        pl.kernel,
        mesh=mesh,
        out_type=jax.ShapeDtypeStruct((batch * msize,), jnp.float32),
        scratch_types=[
            pltpu.VMEM((bpw, dim), jnp.float32),
        ] + [pltpu.VMEM((_CH * msize,), jnp.float32) for _ in range(_NBUF)]
          + [pltpu.SemaphoreType.DMA for _ in range(_NBUF)],
        compiler_params=pltpu.CompilerParams(needs_layout_passes=False),
    )
    def diag_sc(e_hbm, z_hbm, out_hbm, ebuf, *rest):
        bufs = rest[:_NBUF]
        sems = rest[_NBUF:]
        wid = lax.axis_index("s") * nc + lax.axis_index("c")
        base = wid * bpw
        pltpu.sync_copy(e_hbm.at[pl.ds(base, bpw)], ebuf)
        for buf in bufs:
            pltpu.sync_copy(z_hbm, buf)
        handles = [None] * _NBUF
        sizes = [_CH] * (bpw // _CH)
        if bpw % _CH:
            sizes.append(bpw % _CH)
        b0 = 0
        for c, sz in enumerate(sizes):
            s = c % _NBUF
            buf = bufs[s]
            if handles[s] is not None:
                handles[s].wait()
            for m in range(sz):
                b = b0 + m
                for k in range(nchunk):
                    idx = (jnp.arange(lanes, dtype=jnp.int32)
                           + (k * lanes)) * (dim + 1) + (m * msize)
                    vals = jnp.exp(ebuf[b, pl.ds(k * lanes, lanes)])
                    plsc.store_scatter(buf, [idx], vals)
            dst = out_hbm.at[pl.ds((base + b0) * msize, sz * msize)]
            handles[s] = pltpu.async_copy(buf.at[pl.ds(0, sz * msize)], dst,
                                          sems[s])
            b0 += sz
        for h in handles:
            if h is not None:
                h.wait()

    return diag_sc


def kernel(embeddings):
    batch, dim, _ = embeddings.shape
    e2 = embeddings[:, :, 0]
    zeros = jnp.zeros((_CH * dim * dim,), dtype=jnp.float32)
    sc = _make_sc_kernel(batch, dim)
    return sc(e2, zeros).reshape(batch, dim, dim)


# speedup vs baseline: 2.9378x; 2.7846x over previous
import jax
import jax.numpy as jnp
from jax.experimental import pallas as pl
from jax.experimental.pallas import tpu as pltpu

_BLK = 64


def _diag_kernel(e_ref, out_ref):
    dim = e_ref.shape[1]
    vals = jnp.exp(e_ref[...])
    i = jax.lax.broadcasted_iota(jnp.int32, (dim, dim), 0)
    j = jax.lax.broadcasted_iota(jnp.int32, (dim, dim), 1)
    eye = jnp.where(i == j, jnp.float32(1), jnp.float32(0))
    out_ref[...] = vals[:, None, :] * eye[None, :, :]


def kernel(embeddings):
    batch, dim, _ = embeddings.shape
    e2 = embeddings[:, :, 0]
    return pl.pallas_call(
        _diag_kernel,
        grid=(batch // _BLK,),
        in_specs=[pl.BlockSpec((_BLK, dim), lambda b: (b, 0))],
        out_specs=pl.BlockSpec((_BLK, dim, dim), lambda b: (b, 0, 0)),
        out_shape=jax.ShapeDtypeStruct((batch, dim, dim), embeddings.dtype),
        compiler_params=pltpu.CompilerParams(dimension_semantics=("parallel",)),
    )(e2)
